# Initial kernel scaffold; baseline (speedup 1.0000x reference)
import functools

import jax
import jax.numpy as jnp
from jax import lax
from jax.experimental import pallas as pl
from jax.experimental.pallas import tpu as pltpu
from jax.experimental.pallas import tpu_sc as plsc

B, S = 16384, 200
D = 32
N = B * S
NC, NS = 2, 16
NW = NC * NS
NPW = N // NW
IW = 128
CH = 10
C = CH * IW
NCHUNK = NPW // C
ROWS_PER_W = NPW // IW

_mesh = plsc.VectorSubcoreMesh(core_axis_name="c", subcore_axis_name="s")


@functools.partial(
    pl.kernel,
    out_type=jax.ShapeDtypeStruct((N, D), jnp.float32),
    mesh=_mesh,
    scratch_types=[
        pltpu.VMEM((CH, IW), jnp.int32),
        pltpu.VMEM((C, D), jnp.float32),
        pltpu.SemaphoreType.DMA,
    ],
)
def _gather(table_hbm, idx_hbm, out_hbm, idx_v, rows_v, sem):
    wid = lax.axis_index("s") * NC + lax.axis_index("c")
    row_base = wid * ROWS_PER_W
    out_base = wid * NPW

    def chunk(ci, carry):
        pltpu.sync_copy(idx_hbm.at[pl.ds(row_base + ci * CH, CH)], idx_v)
        cps = [
            pltpu.async_copy(
                table_hbm.at[idx_v.at[j]],
                rows_v.at[pl.ds(j * IW, IW)],
                sem,
            )
            for j in range(CH)
        ]
        for cp in cps:
            cp.wait()
        pltpu.sync_copy(rows_v, out_hbm.at[pl.ds(out_base + ci * C, C)])
        return carry

    lax.fori_loop(0, NCHUNK, chunk, 0)


def kernel(seqs, species, table):
    del species
    idx2d = seqs.reshape(-1, IW).astype(jnp.int32)
    out = _gather(table, idx2d)
    return out.reshape(B, S, D)


# speedup vs baseline: 4.8101x; 4.8101x over previous
import functools

import jax
import jax.numpy as jnp
from jax import lax
from jax.experimental import pallas as pl
from jax.experimental.pallas import tpu as pltpu
from jax.experimental.pallas import tpu_sc as plsc

B, S = 16384, 200
D = 32
N = B * S
NC, NS = 2, 16
NW = NC * NS
NPW = N // NW
IW = 128
CH = 8
C = CH * IW
NCHUNK = NPW // C
ROWS_PER_W = NPW // IW

_mesh = plsc.VectorSubcoreMesh(core_axis_name="c", subcore_axis_name="s")


@functools.partial(
    pl.kernel,
    out_type=jax.ShapeDtypeStruct((N, D), jnp.float32),
    mesh=_mesh,
    scratch_types=[
        pltpu.VMEM((CH, IW), jnp.int32),
        pltpu.VMEM((C, D), jnp.float32),
        pltpu.SemaphoreType.DMA,
    ],
    compiler_params=pltpu.CompilerParams(use_tc_tiling_on_sc=False),
)
def _gather(table_hbm, idx_hbm, out_hbm, idx_v, rows_v, sem):
    wid = lax.axis_index("s") * NC + lax.axis_index("c")
    row_base = wid * ROWS_PER_W
    out_base = wid * NPW

    def chunk(ci, carry):
        pltpu.sync_copy(idx_hbm.at[pl.ds(row_base + ci * CH, CH)], idx_v)
        cps = [
            pltpu.async_copy(
                table_hbm.at[idx_v.at[j]],
                rows_v.at[pl.ds(j * IW, IW)],
                sem,
            )
            for j in range(CH)
        ]
        for cp in cps:
            cp.wait()
        pltpu.sync_copy(rows_v, out_hbm.at[pl.ds(out_base + ci * C, C)])
        return carry

    lax.fori_loop(0, NCHUNK, chunk, 0)


def kernel(seqs, species, table):
    del species
    idx2d = seqs.reshape(-1, IW).astype(jnp.int32)
    out = _gather(table, idx2d)
    return out.reshape(B, S, D)


# speedup vs baseline: 5.0348x; 1.0467x over previous
import functools

import jax
import jax.numpy as jnp
from jax import lax
from jax.experimental import pallas as pl
from jax.experimental.pallas import tpu as pltpu
from jax.experimental.pallas import tpu_sc as plsc

B, S = 16384, 200
D = 32
N = B * S
NC, NS = 2, 16
NW = NC * NS
NPW = N // NW
IW = 128
CH = 8
C = CH * IW
NCHUNK = NPW // C
ROWS_PER_W = NPW // IW
NBUF = 2
NSUPER = NCHUNK // NBUF

_mesh = plsc.VectorSubcoreMesh(core_axis_name="c", subcore_axis_name="s")


@functools.partial(
    pl.kernel,
    out_type=jax.ShapeDtypeStruct((N, D), jnp.float32),
    mesh=_mesh,
    scratch_types=[
        pltpu.VMEM((CH, IW), jnp.int32),
        pltpu.VMEM((CH, IW), jnp.int32),
        pltpu.VMEM((C, D), jnp.float32),
        pltpu.VMEM((C, D), jnp.float32),
        pltpu.SemaphoreType.DMA,
        pltpu.SemaphoreType.DMA,
        pltpu.SemaphoreType.DMA,
        pltpu.SemaphoreType.DMA,
        pltpu.SemaphoreType.DMA,
        pltpu.SemaphoreType.DMA,
    ],
    compiler_params=pltpu.CompilerParams(use_tc_tiling_on_sc=False),
)
def _gather(table_hbm, idx_hbm, out_hbm,
            idx0, idx1, rows0, rows1, is0, is1, gs0, gs1, os0, os1):
    idx_v = (idx0, idx1)
    rows_v = (rows0, rows1)
    isem = (is0, is1)
    gsem = (gs0, gs1)
    osem = (os0, os1)

    wid = lax.axis_index("s") * NC + lax.axis_index("c")
    row_base = wid * ROWS_PER_W
    out_base = wid * NPW

    def idx_src(ci):
        return idx_hbm.at[pl.ds(row_base + ci * CH, CH)]

    def out_dst(ci):
        return out_hbm.at[pl.ds(out_base + ci * C, C)]

    for b in range(NBUF):
        pltpu.async_copy(idx_src(b), idx_v[b], isem[b])

    def super_chunk(g, carry):
        for b in range(NBUF):
            ci = g * NBUF + b
            pltpu.make_async_copy(idx_src(ci), idx_v[b], isem[b]).wait()

            @pl.when(g > 0)
            def _():
                pltpu.make_async_copy(rows_v[b], out_dst(ci), osem[b]).wait()

            cps = [
                pltpu.async_copy(
                    table_hbm.at[idx_v[b].at[j]],
                    rows_v[b].at[pl.ds(j * IW, IW)],
                    gsem[b],
                )
                for j in range(CH)
            ]
            for cp in cps:
                cp.wait()

            pltpu.async_copy(rows_v[b], out_dst(ci), osem[b])

            @pl.when(g < NSUPER - 1)
            def _():
                pltpu.async_copy(idx_src(ci + NBUF), idx_v[b], isem[b])
        return carry

    lax.fori_loop(0, NSUPER, super_chunk, 0)

    for b in range(NBUF):
        pltpu.make_async_copy(
            rows_v[b], out_dst(NCHUNK - NBUF + b), osem[b]
        ).wait()


def kernel(seqs, species, table):
    del species
    idx2d = seqs.reshape(-1, IW).astype(jnp.int32)
    out = _gather(table, idx2d)
    return out.reshape(B, S, D)
